# baseline (speedup 1.0000x reference)
import functools

import jax
import jax.numpy as jnp
from jax import lax
from jax.experimental import pallas as pl
from jax.experimental.pallas import tpu as pltpu
from jax.experimental.pallas import tpu_sc as plsc


def _router_tc(x_ref, gwt_ref, bias_ref, scores_ref, choice_ref):
    logits = jnp.dot(x_ref[...], gwt_ref[...], preferred_element_type=jnp.float32)
    scores = jax.nn.sigmoid(logits)
    scores_ref[...] = scores
    choice_ref[...] = scores + bias_ref[0:1, :]


def _make_router(t, h, n_exp):
    return pl.pallas_call(
        _router_tc,
        in_specs=[
            pl.BlockSpec((t, h), lambda: (0, 0)),
            pl.BlockSpec((h, n_exp), lambda: (0, 0)),
            pl.BlockSpec((8, n_exp), lambda: (0, 0)),
        ],
        out_specs=[
            pl.BlockSpec((t, n_exp), lambda: (0, 0)),
            pl.BlockSpec((t, n_exp), lambda: (0, 0)),
        ],
        out_shape=[
            jax.ShapeDtypeStruct((t, n_exp), jnp.float32),
            jax.ShapeDtypeStruct((t, n_exp), jnp.float32),
        ],
    )


def _topk_sc_body(scores_hbm, choice_hbm, zeros_hbm, w_hbm,
                  sc_v, ch_v, w_v, *, n_exp, top_k, n_groups):
    info = plsc.get_sparse_core_info()
    nc = info.num_cores
    wid = lax.axis_index("s") * nc + lax.axis_index("c")
    grp = 16 * n_exp

    @pl.when(wid < n_groups)
    def _work():
        base = wid * grp
        pltpu.sync_copy(scores_hbm.at[pl.ds(base, grp)], sc_v)
        pltpu.sync_copy(choice_hbm.at[pl.ds(base, grp)], ch_v)
        pltpu.sync_copy(zeros_hbm.at[pl.ds(base, grp)], w_v)

        rowoff = lax.iota(jnp.int32, 16) * n_exp
        neg_inf = jnp.full((16,), -jnp.inf, dtype=jnp.float32)
        zero_i = jnp.zeros((16,), dtype=jnp.int32)
        ms = [neg_inf] * top_k
        is_ = [zero_i] * top_k
        for e in range(n_exp):
            col = jnp.full((16,), e, dtype=jnp.int32)
            v = plsc.load_gather(ch_v, [rowoff + col])
            vi = col
            for j in range(top_k):
                gt = v > ms[j]
                nm = jnp.where(gt, v, ms[j])
                nv = jnp.where(gt, ms[j], v)
                ni = jnp.where(gt, vi, is_[j])
                nvi = jnp.where(gt, is_[j], vi)
                ms[j], v, is_[j], vi = nm, nv, ni, nvi
        svals = [plsc.load_gather(sc_v, [rowoff + is_[j]]) for j in range(top_k)]
        denom = svals[0]
        for j in range(1, top_k):
            denom = denom + svals[j]
        denom = denom + 1e-20
        for j in range(top_k):
            plsc.store_scatter(w_v, [rowoff + is_[j]], svals[j] / denom)
        pltpu.sync_copy(w_v, w_hbm.at[pl.ds(base, grp)])


def _make_topk_sc(t, n_exp, top_k):
    mesh = plsc.VectorSubcoreMesh(core_axis_name="c", subcore_axis_name="s")
    body = functools.partial(_topk_sc_body, n_exp=n_exp, top_k=top_k,
                             n_groups=t // 16)
    return functools.partial(
        pl.kernel, mesh=mesh,
        compiler_params=pltpu.CompilerParams(needs_layout_passes=False),
        out_type=jax.ShapeDtypeStruct((t * n_exp,), jnp.float32),
        scratch_types=[
            pltpu.VMEM((16 * n_exp,), jnp.float32),
            pltpu.VMEM((16 * n_exp,), jnp.float32),
            pltpu.VMEM((16 * n_exp,), jnp.float32),
        ],
    )(body)


def _moe_body(x_ref, w_ref, wg_ref, wu_ref, wd_ref,
              sg_ref, su_ref, sd_ref, out_ref, xbf_s):
    s = pl.program_id(0)
    t = x_ref.shape[0]

    @pl.when(s == 0)
    def _init():
        xbf_s[...] = x_ref[...].astype(jnp.bfloat16)
        out_ref[...] = jnp.zeros_like(out_ref)

    xbf = xbf_s[...]

    def wcol(idx):
        col = jax.lax.broadcasted_iota(jnp.int32, w_ref.shape, 1)
        return jnp.sum(jnp.where(col == idx, w_ref[...], 0.0), axis=1, keepdims=True)

    @pl.when(s == 0)
    def _shared():
        g = jnp.dot(xbf, sg_ref[...].astype(jnp.bfloat16), preferred_element_type=jnp.float32)
        u = jnp.dot(xbf, su_ref[...].astype(jnp.bfloat16), preferred_element_type=jnp.float32)
        h = g * jax.nn.sigmoid(g) * u
        out_ref[...] += jnp.dot(h.astype(jnp.bfloat16), sd_ref[...].astype(jnp.bfloat16),
                                preferred_element_type=jnp.float32)

    @pl.when(s >= 1)
    def _routed():
        e0 = (s - 1) * 4
        nb, _, fb = wg_ref.shape
        hs = []
        for j in range(nb):
            g = jnp.dot(xbf, wg_ref[j].astype(jnp.bfloat16),
                        preferred_element_type=jnp.float32)
            u = jnp.dot(xbf, wu_ref[j].astype(jnp.bfloat16),
                        preferred_element_type=jnp.float32)
            h = g * jax.nn.sigmoid(g) * u
            hs.append((wcol(e0 + j) * h).astype(jnp.bfloat16))
        h_cat = jnp.concatenate(hs, axis=1)
        wd_flat = wd_ref[...].reshape(nb * fb, wd_ref.shape[2])
        out_ref[...] += jnp.dot(h_cat, wd_flat.astype(jnp.bfloat16),
                                preferred_element_type=jnp.float32)


def kernel(hidden_states, positions, gate_w, corr_bias, w_gate, w_up, w_down, sg, su, sd):
    del positions
    t, h = hidden_states.shape
    n_exp, _, f = w_gate.shape
    sf = sg.shape[1]
    top_k = 8
    grid = (1 + n_exp // 4,)

    gwt = gate_w.T
    bias2d = jnp.broadcast_to(corr_bias[None, :], (8, n_exp))

    scores, choice = _make_router(t, h, n_exp)(hidden_states, gwt, bias2d)
    zeros = jnp.zeros((t * n_exp,), jnp.float32)
    w_flat = _make_topk_sc(t, n_exp, top_k)(
        scores.reshape(t * n_exp), choice.reshape(t * n_exp), zeros)
    w_comb = w_flat.reshape(t, n_exp)

    routed_idx = lambda s: (jnp.maximum(s - 1, 0), 0, 0)

    out = pl.pallas_call(
        _moe_body,
        grid=grid,
        in_specs=[
            pl.BlockSpec((t, h), lambda s: (0, 0)),
            pl.BlockSpec((t, n_exp), lambda s: (0, 0)),
            pl.BlockSpec((4, h, f), routed_idx),
            pl.BlockSpec((4, h, f), routed_idx),
            pl.BlockSpec((4, f, h), routed_idx),
            pl.BlockSpec((h, sf), lambda s: (0, 0)),
            pl.BlockSpec((h, sf), lambda s: (0, 0)),
            pl.BlockSpec((sf, h), lambda s: (0, 0)),
        ],
        out_specs=pl.BlockSpec((t, h), lambda s: (0, 0)),
        out_shape=jax.ShapeDtypeStruct((t, h), jnp.float32),
        scratch_shapes=[
            pltpu.VMEM((t, h), jnp.bfloat16),
        ],
        compiler_params=pltpu.CompilerParams(
            dimension_semantics=("arbitrary",),
        ),
    )(hidden_states, w_comb, w_gate, w_up, w_down, sg, su, sd)
    return out


# speedup vs baseline: 1.3209x; 1.3209x over previous
import functools

import jax
import jax.numpy as jnp
from jax.experimental import pallas as pl
from jax.experimental.pallas import tpu as pltpu


def _moe_body(x_ref, gwt_ref, bias_ref, wg_ref, wu_ref, wd_ref,
              sg_ref, su_ref, sd_ref, out_ref, w_s, xbf_s, *, n_exp, top_k):
    s = pl.program_id(0)
    t = x_ref.shape[0]

    @pl.when(s == 0)
    def _init():
        x = x_ref[...]
        xbf_s[...] = x.astype(jnp.bfloat16)
        out_ref[...] = jnp.zeros_like(out_ref)
        logits = jnp.dot(x, gwt_ref[...], preferred_element_type=jnp.float32)
        scores = jax.nn.sigmoid(logits)
        choice = scores + bias_ref[0:1, :]
        col = jax.lax.broadcasted_iota(jnp.int32, (t, n_exp), 1)
        masked = choice
        selected = jnp.zeros((t, n_exp), dtype=jnp.bool_)
        for _ in range(top_k):
            m = jnp.max(masked, axis=1, keepdims=True)
            cand = jnp.where(masked == m, col, n_exp)
            amin = jnp.min(cand, axis=1, keepdims=True)
            sel = col == amin
            selected = jnp.logical_or(selected, sel)
            masked = jnp.where(sel, -jnp.inf, masked)
        kept = jnp.where(selected, scores, 0.0)
        denom = jnp.sum(kept, axis=1, keepdims=True) + 1e-20
        w_norm = kept / denom
        for e in range(n_exp):
            w_s[e] = w_norm[:, e:e + 1]

    xbf = xbf_s[...]

    def ffn(wg, wu, wd):
        g = jnp.dot(xbf, wg.astype(jnp.bfloat16), preferred_element_type=jnp.float32)
        u = jnp.dot(xbf, wu.astype(jnp.bfloat16), preferred_element_type=jnp.float32)
        h = g * jax.nn.sigmoid(g) * u
        return jnp.dot(h.astype(jnp.bfloat16), wd.astype(jnp.bfloat16),
                       preferred_element_type=jnp.float32)

    def wcol(idx):
        return w_s[pl.ds(idx, 1)][0]

    @pl.when(s == 0)
    def _shared():
        out_ref[...] += ffn(sg_ref[...], su_ref[...], sd_ref[...])

    @pl.when(s >= 1)
    def _routed():
        e0 = (s - 1) * 4
        nb, _, fb = wg_ref.shape
        hs = []
        for j in range(nb):
            g = jnp.dot(xbf, wg_ref[j].astype(jnp.bfloat16),
                        preferred_element_type=jnp.float32)
            u = jnp.dot(xbf, wu_ref[j].astype(jnp.bfloat16),
                        preferred_element_type=jnp.float32)
            h = g * jax.nn.sigmoid(g) * u
            hs.append((wcol(e0 + j) * h).astype(jnp.bfloat16))
        h_cat = jnp.concatenate(hs, axis=1)
        wd_flat = wd_ref[...].reshape(nb * fb, wd_ref.shape[2])
        out_ref[...] += jnp.dot(h_cat, wd_flat.astype(jnp.bfloat16),
                                preferred_element_type=jnp.float32)


def kernel(hidden_states, positions, gate_w, corr_bias, w_gate, w_up, w_down, sg, su, sd):
    del positions
    t, h = hidden_states.shape
    n_exp, _, f = w_gate.shape
    sf = sg.shape[1]
    top_k = 8
    grid = (1 + n_exp // 4,)

    gwt = gate_w.T
    bias2d = jnp.broadcast_to(corr_bias[None, :], (8, n_exp))

    routed_idx = lambda s: (jnp.maximum(s - 1, 0), 0, 0)

    body = functools.partial(_moe_body, n_exp=n_exp, top_k=top_k)

    out = pl.pallas_call(
        body,
        grid=grid,
        in_specs=[
            pl.BlockSpec((t, h), lambda s: (0, 0)),
            pl.BlockSpec((h, n_exp), lambda s: (0, 0)),
            pl.BlockSpec((8, n_exp), lambda s: (0, 0)),
            pl.BlockSpec((4, h, f), routed_idx),
            pl.BlockSpec((4, h, f), routed_idx),
            pl.BlockSpec((4, f, h), routed_idx),
            pl.BlockSpec((h, sf), lambda s: (0, 0)),
            pl.BlockSpec((h, sf), lambda s: (0, 0)),
            pl.BlockSpec((sf, h), lambda s: (0, 0)),
        ],
        out_specs=pl.BlockSpec((t, h), lambda s: (0, 0)),
        out_shape=jax.ShapeDtypeStruct((t, h), jnp.float32),
        scratch_shapes=[
            pltpu.VMEM((n_exp, t, 1), jnp.float32),
            pltpu.VMEM((t, h), jnp.bfloat16),
        ],
        compiler_params=pltpu.CompilerParams(
            dimension_semantics=("arbitrary",),
        ),
    )(hidden_states, gwt, bias2d, w_gate, w_up, w_down, sg, su, sd)
    return out


# speedup vs baseline: 1.3354x; 1.0110x over previous
import functools

import jax
import jax.numpy as jnp
from jax.experimental import pallas as pl
from jax.experimental.pallas import tpu as pltpu


def _moe_body(x_ref, gwt_ref, bias_ref, wg_ref, wu_ref, wd_ref,
              sg_ref, su_ref, sd_ref, out_ref, w_s, xbf_s, *, n_exp, top_k):
    s = pl.program_id(0)
    t = x_ref.shape[0]

    @pl.when(s == 0)
    def _init():
        x = x_ref[...]
        xbf_s[...] = x.astype(jnp.bfloat16)
        out_ref[...] = jnp.zeros_like(out_ref)
        logits = jnp.dot(x, gwt_ref[...], preferred_element_type=jnp.float32)
        scores = jax.nn.sigmoid(logits)
        choice = scores + bias_ref[0:1, :]
        col = jax.lax.broadcasted_iota(jnp.int32, (t, n_exp), 1)
        masked = choice
        selected = jnp.zeros((t, n_exp), dtype=jnp.bool_)
        for _ in range(top_k):
            m = jnp.max(masked, axis=1, keepdims=True)
            cand = jnp.where(masked == m, col, n_exp)
            amin = jnp.min(cand, axis=1, keepdims=True)
            sel = col == amin
            selected = jnp.logical_or(selected, sel)
            masked = jnp.where(sel, -jnp.inf, masked)
        kept = jnp.where(selected, scores, 0.0)
        denom = jnp.sum(kept, axis=1, keepdims=True) + 1e-20
        w_s[...] = kept / denom

    xbf = xbf_s[...]

    def ffn(wg, wu, wd):
        g = jnp.dot(xbf, wg.astype(jnp.bfloat16), preferred_element_type=jnp.float32)
        u = jnp.dot(xbf, wu.astype(jnp.bfloat16), preferred_element_type=jnp.float32)
        h = g * jax.nn.sigmoid(g) * u
        return jnp.dot(h.astype(jnp.bfloat16), wd.astype(jnp.bfloat16),
                       preferred_element_type=jnp.float32)

    def wcol(idx):
        col = jax.lax.broadcasted_iota(jnp.int32, w_s.shape, 1)
        return jnp.sum(jnp.where(col == idx, w_s[...], 0.0), axis=1, keepdims=True)

    @pl.when(s == 0)
    def _shared():
        out_ref[...] += ffn(sg_ref[...], su_ref[...], sd_ref[...])

    @pl.when(s >= 1)
    def _routed():
        e0 = (s - 1) * 4
        nb, _, fb = wg_ref.shape
        gu_cat = jnp.concatenate(
            [wg_ref[j].astype(jnp.bfloat16) for j in range(nb)]
            + [wu_ref[j].astype(jnp.bfloat16) for j in range(nb)], axis=1)
        gu = jnp.dot(xbf, gu_cat, preferred_element_type=jnp.float32)
        hs = []
        for j in range(nb):
            g = gu[:, j * fb:(j + 1) * fb]
            u = gu[:, (nb + j) * fb:(nb + j + 1) * fb]
            h = g * jax.nn.sigmoid(g) * u
            hs.append((wcol(e0 + j) * h).astype(jnp.bfloat16))
        h_cat = jnp.concatenate(hs, axis=1)
        wd_flat = wd_ref[...].reshape(nb * fb, wd_ref.shape[2])
        out_ref[...] += jnp.dot(h_cat, wd_flat.astype(jnp.bfloat16),
                                preferred_element_type=jnp.float32)


def kernel(hidden_states, positions, gate_w, corr_bias, w_gate, w_up, w_down, sg, su, sd):
    del positions
    t, h = hidden_states.shape
    n_exp, _, f = w_gate.shape
    sf = sg.shape[1]
    top_k = 8
    grid = (1 + n_exp // 4,)

    gwt = gate_w.T
    bias2d = jnp.broadcast_to(corr_bias[None, :], (8, n_exp))

    routed_idx = lambda s: (jnp.maximum(s - 1, 0), 0, 0)

    body = functools.partial(_moe_body, n_exp=n_exp, top_k=top_k)

    out = pl.pallas_call(
        body,
        grid=grid,
        in_specs=[
            pl.BlockSpec((t, h), lambda s: (0, 0)),
            pl.BlockSpec((h, n_exp), lambda s: (0, 0)),
            pl.BlockSpec((8, n_exp), lambda s: (0, 0)),
            pl.BlockSpec((4, h, f), routed_idx),
            pl.BlockSpec((4, h, f), routed_idx),
            pl.BlockSpec((4, f, h), routed_idx),
            pl.BlockSpec((h, sf), lambda s: (0, 0)),
            pl.BlockSpec((h, sf), lambda s: (0, 0)),
            pl.BlockSpec((sf, h), lambda s: (0, 0)),
        ],
        out_specs=pl.BlockSpec((t, h), lambda s: (0, 0)),
        out_shape=jax.ShapeDtypeStruct((t, h), jnp.float32),
        scratch_shapes=[
            pltpu.VMEM((t, n_exp), jnp.float32),
            pltpu.VMEM((t, h), jnp.bfloat16),
        ],
        compiler_params=pltpu.CompilerParams(
            dimension_semantics=("arbitrary",),
        ),
    )(hidden_states, gwt, bias2d, w_gate, w_up, w_down, sg, su, sd)
    return out


# speedup vs baseline: 1.3383x; 1.0021x over previous
import functools

import jax
import jax.numpy as jnp
from jax.experimental import pallas as pl
from jax.experimental.pallas import tpu as pltpu


def _moe_body(x_ref, gwt_ref, bias_ref, wg_ref, wu_ref, wd_ref,
              sg_ref, su_ref, sd_ref, out_ref, w_s, xbf_s, *, n_exp, top_k):
    s = pl.program_id(0)
    t = x_ref.shape[0]

    @pl.when(s == 0)
    def _init():
        x = x_ref[...]
        xbf_s[...] = x.astype(jnp.bfloat16)
        out_ref[...] = jnp.zeros_like(out_ref)
        logits = jnp.dot(x, gwt_ref[...], preferred_element_type=jnp.float32)
        scores = jax.nn.sigmoid(logits)
        choice = scores + bias_ref[0:1, :]
        col = jax.lax.broadcasted_iota(jnp.int32, (t, n_exp), 1)
        masked = choice
        selected = jnp.zeros((t, n_exp), dtype=jnp.bool_)
        for _ in range(top_k):
            m = jnp.max(masked, axis=1, keepdims=True)
            cand = jnp.where(masked == m, col, n_exp)
            amin = jnp.min(cand, axis=1, keepdims=True)
            sel = col == amin
            selected = jnp.logical_or(selected, sel)
            masked = jnp.where(sel, -jnp.inf, masked)
        kept = jnp.where(selected, scores, 0.0)
        denom = jnp.sum(kept, axis=1, keepdims=True) + 1e-20
        w_s[...] = kept / denom

    xbf = xbf_s[...]

    def ffn(wg, wu, wd):
        g = jnp.dot(xbf, wg.astype(jnp.bfloat16), preferred_element_type=jnp.float32)
        u = jnp.dot(xbf, wu.astype(jnp.bfloat16), preferred_element_type=jnp.float32)
        h = g * jax.nn.sigmoid(g) * u
        return jnp.dot(h.astype(jnp.bfloat16), wd.astype(jnp.bfloat16),
                       preferred_element_type=jnp.float32)

    def wcol(idx):
        col = jax.lax.broadcasted_iota(jnp.int32, w_s.shape, 1)
        return jnp.sum(jnp.where(col == idx, w_s[...], 0.0), axis=1, keepdims=True)

    @pl.when(s == 0)
    def _shared():
        out_ref[...] += ffn(sg_ref[...], su_ref[...], sd_ref[...])

    @pl.when(s >= 1)
    def _routed():
        e0 = (s - 1) * 4
        nb, _, fb = wg_ref.shape
        hs = []
        for j in range(nb):
            g = jnp.dot(xbf, wg_ref[j].astype(jnp.bfloat16),
                        preferred_element_type=jnp.float32)
            u = jnp.dot(xbf, wu_ref[j].astype(jnp.bfloat16),
                        preferred_element_type=jnp.float32)
            h = g * jax.nn.sigmoid(g) * u
            hs.append((wcol(e0 + j) * h).astype(jnp.bfloat16))
        h_cat = jnp.concatenate(hs, axis=1)
        wd_flat = wd_ref[...].reshape(nb * fb, wd_ref.shape[2])
        out_ref[...] += jnp.dot(h_cat, wd_flat.astype(jnp.bfloat16),
                                preferred_element_type=jnp.float32)


def kernel(hidden_states, positions, gate_w, corr_bias, w_gate, w_up, w_down, sg, su, sd):
    del positions
    t, h = hidden_states.shape
    n_exp, _, f = w_gate.shape
    sf = sg.shape[1]
    top_k = 8
    grid = (1 + n_exp // 4,)

    gwt = gate_w.T
    bias2d = jnp.broadcast_to(corr_bias[None, :], (8, n_exp))

    routed_idx = lambda s: (jnp.maximum(s - 1, 0), 0, 0)

    body = functools.partial(_moe_body, n_exp=n_exp, top_k=top_k)

    out = pl.pallas_call(
        body,
        grid=grid,
        in_specs=[
            pl.BlockSpec((t, h), lambda s: (0, 0)),
            pl.BlockSpec((h, n_exp), lambda s: (0, 0)),
            pl.BlockSpec((8, n_exp), lambda s: (0, 0)),
            pl.BlockSpec((4, h, f), routed_idx),
            pl.BlockSpec((4, h, f), routed_idx),
            pl.BlockSpec((4, f, h), routed_idx),
            pl.BlockSpec((h, sf), lambda s: (0, 0)),
            pl.BlockSpec((h, sf), lambda s: (0, 0)),
            pl.BlockSpec((sf, h), lambda s: (0, 0)),
        ],
        out_specs=pl.BlockSpec((t, h), lambda s: (0, 0)),
        out_shape=jax.ShapeDtypeStruct((t, h), jnp.float32),
        scratch_shapes=[
            pltpu.VMEM((t, n_exp), jnp.float32),
            pltpu.VMEM((t, h), jnp.bfloat16),
        ],
        compiler_params=pltpu.CompilerParams(
            dimension_semantics=("arbitrary",),
        ),
    )(hidden_states, gwt, bias2d, w_gate, w_up, w_down, sg, su, sd)
    return out
